# Initial kernel scaffold; baseline (speedup 1.0000x reference)
import jax
import jax.numpy as jnp
from jax.experimental import pallas as pl


def kernel(inputs, adj, weight):
    raise NotImplementedError("write your pallas kernel here")


# speedup vs baseline: 1.0376x; 1.0376x over previous
import jax
import jax.numpy as jnp
from jax.experimental import pallas as pl
from jax.experimental.pallas import tpu as pltpu


def _gcn_body(adj_ref, x_ref, w_ref, out_ref):
    y = jnp.dot(adj_ref[...], x_ref[...], preferred_element_type=jnp.float32)
    z = jnp.dot(y, w_ref[...], preferred_element_type=jnp.float32)
    out_ref[...] = jnp.maximum(z, 0.0)


def kernel(inputs, adj, weight):
    n, d_in = inputs.shape
    d_out = weight.shape[1]
    bm = 400
    grid = (n // bm,)
    return pl.pallas_call(
        _gcn_body,
        grid=grid,
        in_specs=[
            pl.BlockSpec((bm, n), lambda i: (i, 0)),
            pl.BlockSpec((n, d_in), lambda i: (0, 0)),
            pl.BlockSpec((d_in, d_out), lambda i: (0, 0)),
        ],
        out_specs=pl.BlockSpec((bm, d_out), lambda i: (i, 0)),
        out_shape=jax.ShapeDtypeStruct((n, d_out), jnp.float32),
        compiler_params=pltpu.CompilerParams(
            dimension_semantics=("arbitrary",),
        ),
    )(adj, inputs, weight)
